# baseline (speedup 1.0000x reference)
import functools

import numpy as np
import jax
import jax.numpy as jnp
from jax import lax
from jax.experimental import pallas as pl
from jax.experimental.pallas import tpu as pltpu
from jax.experimental.pallas import tpu_sc as plsc

NC = 2
NS = 16
NW = NC * NS
K = 128
FRAC0 = 0.588


def _sc_agg(h, edge3d, zblk, npad, nch8, cmax8, t0c, b0, r0, b1, r1):
    n, d = h.shape
    rpt = npad // NS

    mesh = plsc.VectorSubcoreMesh(
        core_axis_name="c", subcore_axis_name="s",
        num_cores=NC, num_subcores=NS)

    @functools.partial(
        pl.kernel,
        out_type=jax.ShapeDtypeStruct((NC, npad, d), jnp.float32),
        mesh=mesh,
        scratch_types=[
            pltpu.VMEM((cmax8, K), jnp.int32),
            pltpu.VMEM((cmax8, K), jnp.int32),
            pltpu.VMEM((K, d), jnp.float32),
            pltpu.VMEM_SHARED((npad, d), jnp.float32),
            pltpu.SemaphoreType.DMA,
        ],
    )
    def agg(h_hbm, e_hbm, z_hbm, out_hbm, src_v, dst_v, rows_v, acc, sem):
        c = lax.axis_index("c")
        s = lax.axis_index("s")
        pltpu.sync_copy(z_hbm, acc.at[pl.ds(s * rpt, rpt)])
        b = jnp.where(c == 0, b0, b1)
        r = jnp.where(c == 0, r0, r1)
        cw = b + (s < r).astype(jnp.int32)
        off = (jnp.where(c == 0, 0, t0c) + s * b
               + jnp.minimum(s, r))
        astart = jnp.minimum((off // 8) * 8, nch8 - cmax8)
        rem = off - astart
        pltpu.sync_copy(e_hbm.at[0].at[pl.ds(astart, cmax8)], src_v)
        pltpu.sync_copy(e_hbm.at[1].at[pl.ds(astart, cmax8)], dst_v)
        plsc.subcore_barrier()

        def chunk(j, carry):
            pltpu.async_copy(h_hbm.at[src_v.at[j + rem]], rows_v, sem).wait()
            pltpu.sync_copy(rows_v, acc.at[dst_v.at[j + rem]], add=True)
            return carry

        lax.fori_loop(0, cw, chunk, 0, unroll=False)
        plsc.subcore_barrier()
        pltpu.sync_copy(acc.at[pl.ds(s * rpt, rpt)],
                        out_hbm.at[c].at[pl.ds(s * rpt, rpt)])

    return agg(h, edge3d, zblk)


def _mlp_body(n, h_ref, p_ref, wa_ref, ba_ref, wb_ref, bb_ref,
              y_ref, sum_ref, sq_ref):
    i = pl.program_id(0)
    z = h_ref[...] + p_ref[0] + p_ref[1]
    z = jnp.maximum(
        jnp.dot(z, wa_ref[...], preferred_element_type=jnp.float32)
        + ba_ref[...], 0.0)
    y = (jnp.dot(z, wb_ref[...], preferred_element_type=jnp.float32)
         + bb_ref[...])
    y_ref[...] = y

    @pl.when(i == 0)
    def _():
        sum_ref[...] = jnp.zeros_like(sum_ref)
        sq_ref[...] = jnp.zeros_like(sq_ref)

    sum_ref[...] += jnp.sum(y, axis=0, keepdims=True)
    sq_ref[...] += jnp.sum(y * y, axis=0, keepdims=True)


def _mlp_stats(h, parts, wa, ba, wb, bb, blk):
    n, d = h.shape
    hh = wb.shape[1]
    nblk = n // blk
    return pl.pallas_call(
        functools.partial(_mlp_body, n),
        grid=(nblk,),
        in_specs=[
            pl.BlockSpec((blk, d), lambda i: (i, 0)),
            pl.BlockSpec((NC, blk, d), lambda i: (0, i, 0)),
            pl.BlockSpec((d, hh), lambda i: (0, 0)),
            pl.BlockSpec((1, hh), lambda i: (0, 0)),
            pl.BlockSpec((hh, hh), lambda i: (0, 0)),
            pl.BlockSpec((1, hh), lambda i: (0, 0)),
        ],
        out_specs=[
            pl.BlockSpec((blk, hh), lambda i: (i, 0)),
            pl.BlockSpec((1, hh), lambda i: (0, 0)),
            pl.BlockSpec((1, hh), lambda i: (0, 0)),
        ],
        out_shape=[
            jax.ShapeDtypeStruct((n, hh), jnp.float32),
            jax.ShapeDtypeStruct((1, hh), jnp.float32),
            jax.ShapeDtypeStruct((1, hh), jnp.float32),
        ],
    )(h, parts, wa, ba, wb, bb)


def _bn_body(n, y_ref, sum_ref, sq_ref, g_ref, be_ref, o_ref):
    mean = sum_ref[...] / n
    var = sq_ref[...] / n - mean * mean
    scale = lax.rsqrt(var + 1e-5) * g_ref[...]
    o_ref[...] = jnp.maximum((y_ref[...] - mean) * scale + be_ref[...], 0.0)


def _bn_relu(y, s, q, g, be, blk):
    n, hh = y.shape
    return pl.pallas_call(
        functools.partial(_bn_body, n),
        grid=(n // blk,),
        in_specs=[
            pl.BlockSpec((blk, hh), lambda i: (i, 0)),
            pl.BlockSpec((1, hh), lambda i: (0, 0)),
            pl.BlockSpec((1, hh), lambda i: (0, 0)),
            pl.BlockSpec((1, hh), lambda i: (0, 0)),
            pl.BlockSpec((1, hh), lambda i: (0, 0)),
        ],
        out_specs=pl.BlockSpec((blk, hh), lambda i: (i, 0)),
        out_shape=jax.ShapeDtypeStruct((n, hh), jnp.float32),
    )(y, s, q, g, be)


def _pool_body(n, g, nblk, y_ref, sum_ref, sq_ref, g2_ref, be_ref, b_ref,
               w5_ref, b5_ref, o_ref, acc_ref, cnt_ref):
    i = pl.program_id(0)

    @pl.when(i == 0)
    def _():
        acc_ref[...] = jnp.zeros_like(acc_ref)
        cnt_ref[...] = jnp.zeros_like(cnt_ref)

    mean = sum_ref[...] / n
    var = sq_ref[...] / n - mean * mean
    scale = lax.rsqrt(var + 1e-5) * g2_ref[...]
    h = jnp.maximum((y_ref[...] - mean) * scale + be_ref[...], 0.0)

    mask = (b_ref[...] == lax.broadcasted_iota(jnp.int32, (1, g), 1)
            ).astype(jnp.float32)
    acc_ref[...] += lax.dot_general(mask, h, (((0,), (0,)), ((), ())))
    cnt_ref[...] += jnp.sum(mask, axis=0, keepdims=True)

    @pl.when(i == nblk - 1)
    def _():
        pooled = acc_ref[...] / jnp.maximum(cnt_ref[...], 1.0).reshape(g, 1)
        logits = (jnp.dot(pooled, w5_ref[...],
                          preferred_element_type=jnp.float32) + b5_ref[...])
        m = jnp.max(logits, axis=1, keepdims=True)
        lse = jnp.log(jnp.sum(jnp.exp(logits - m), axis=1, keepdims=True)) + m
        o_ref[...] = logits - lse


def _pool_classify(y, s, q, g2, be2, batch2d, w5p, b5p, g, blk):
    n, hh = y.shape
    cp = w5p.shape[1]
    nblk = n // blk
    return pl.pallas_call(
        functools.partial(_pool_body, n, g, nblk),
        grid=(nblk,),
        in_specs=[
            pl.BlockSpec((blk, hh), lambda i: (i, 0)),
            pl.BlockSpec((1, hh), lambda i: (0, 0)),
            pl.BlockSpec((1, hh), lambda i: (0, 0)),
            pl.BlockSpec((1, hh), lambda i: (0, 0)),
            pl.BlockSpec((1, hh), lambda i: (0, 0)),
            pl.BlockSpec((blk, 1), lambda i: (i, 0)),
            pl.BlockSpec((hh, cp), lambda i: (0, 0)),
            pl.BlockSpec((1, cp), lambda i: (0, 0)),
        ],
        out_specs=pl.BlockSpec((g, cp), lambda i: (0, 0)),
        out_shape=jax.ShapeDtypeStruct((g, cp), jnp.float32),
        scratch_shapes=[
            pltpu.VMEM((g, hh), jnp.float32),
            pltpu.VMEM((1, g), jnp.float32),
        ],
    )(y, s, q, g2, be2, batch2d, w5p, b5p)


def kernel(x, edge_index, batch, W1, b1, W2, b2, g1, be1, W3, b3, W4, b4,
           g2, be2, W5, b5):
    n, d = x.shape
    e = edge_index.shape[1]
    g = 64
    blk = 2000

    nch = -(-e // K)
    nch8 = -(-nch // 8) * 8
    if e != nch8 * K:
        pad = nch8 * K - e
        edge_index = jnp.concatenate(
            [edge_index,
             jnp.stack([jnp.zeros((pad,), jnp.int32),
                        jnp.full((pad,), n, jnp.int32)])], axis=1)
    edge3d = edge_index.reshape(2, nch8, K)
    t0c = min(nch - 1, max(1, int(round(FRAC0 * nch))))
    cb0, cr0 = divmod(t0c, NS)
    cb1, cr1 = divmod(nch - t0c, NS)
    cmax8 = -(-(max(cb0, cb1) + 1 + 7) // 8) * 8
    npad = ((n + NS * 8 - 1) // (NS * 8)) * (NS * 8)
    if npad == n:
        npad += NS * 8
    zblk = jnp.zeros((npad // NS, d), jnp.float32)

    batch2d = batch.reshape(n, 1)
    cpad = 128
    c = W5.shape[1]
    w5p = jnp.pad(W5, ((0, 0), (0, cpad - c)))
    b5p = jnp.pad(b5, (0, cpad - c), constant_values=-1e30).reshape(1, cpad)

    p1 = _sc_agg(x, edge3d, zblk, npad, nch8, cmax8, t0c, cb0, cr0, cb1, cr1)
    y1, s1, q1 = _mlp_stats(x, p1, W1, b1.reshape(1, -1),
                            W2, b2.reshape(1, -1), blk)
    h1 = _bn_relu(y1, s1, q1, g1.reshape(1, -1), be1.reshape(1, -1), blk)

    p2 = _sc_agg(h1, edge3d, zblk, npad, nch8, cmax8, t0c, cb0, cr0, cb1, cr1)
    y2, s2, q2 = _mlp_stats(h1, p2, W3, b3.reshape(1, -1),
                            W4, b4.reshape(1, -1), blk)

    out = _pool_classify(y2, s2, q2, g2.reshape(1, -1), be2.reshape(1, -1),
                         batch2d, w5p, b5p, g, blk)
    return out[:, :c]


# speedup vs baseline: 1.0343x; 1.0343x over previous
import functools

import numpy as np
import jax
import jax.numpy as jnp
from jax import lax
from jax.experimental import pallas as pl
from jax.experimental.pallas import tpu as pltpu
from jax.experimental.pallas import tpu_sc as plsc

NC = 2
NS = 16
NW = NC * NS
K = 128
FRAC0 = 0.560


def _sc_agg(h, edge3d, zblk, npad, nch8, cmax8, t0c, b0, r0, b1, r1):
    n, d = h.shape
    rpt = npad // NS

    mesh = plsc.VectorSubcoreMesh(
        core_axis_name="c", subcore_axis_name="s",
        num_cores=NC, num_subcores=NS)

    @functools.partial(
        pl.kernel,
        out_type=jax.ShapeDtypeStruct((NC, npad, d), jnp.float32),
        mesh=mesh,
        scratch_types=[
            pltpu.VMEM((cmax8, K), jnp.int32),
            pltpu.VMEM((cmax8, K), jnp.int32),
            pltpu.VMEM((K, d), jnp.float32),
            pltpu.VMEM_SHARED((npad, d), jnp.float32),
            pltpu.SemaphoreType.DMA,
        ],
    )
    def agg(h_hbm, e_hbm, z_hbm, out_hbm, src_v, dst_v, rows_v, acc, sem):
        c = lax.axis_index("c")
        s = lax.axis_index("s")
        pltpu.sync_copy(z_hbm, acc.at[pl.ds(s * rpt, rpt)])
        b = jnp.where(c == 0, b0, b1)
        r = jnp.where(c == 0, r0, r1)
        cw = b + (s < r).astype(jnp.int32)
        off = (jnp.where(c == 0, 0, t0c) + s * b
               + jnp.minimum(s, r))
        astart = jnp.minimum((off // 8) * 8, nch8 - cmax8)
        rem = off - astart
        pltpu.sync_copy(e_hbm.at[0].at[pl.ds(astart, cmax8)], src_v)
        pltpu.sync_copy(e_hbm.at[1].at[pl.ds(astart, cmax8)], dst_v)
        plsc.subcore_barrier()

        def chunk(j, carry):
            pltpu.async_copy(h_hbm.at[src_v.at[j + rem]], rows_v, sem).wait()
            pltpu.sync_copy(rows_v, acc.at[dst_v.at[j + rem]], add=True)
            return carry

        lax.fori_loop(0, cw, chunk, 0, unroll=False)
        plsc.subcore_barrier()
        pltpu.sync_copy(acc.at[pl.ds(s * rpt, rpt)],
                        out_hbm.at[c].at[pl.ds(s * rpt, rpt)])

    return agg(h, edge3d, zblk)


def _mlp_body(n, h_ref, p_ref, wa_ref, ba_ref, wb_ref, bb_ref,
              y_ref, sum_ref, sq_ref):
    i = pl.program_id(0)
    z = h_ref[...] + p_ref[0] + p_ref[1]
    z = jnp.maximum(
        jnp.dot(z, wa_ref[...], preferred_element_type=jnp.float32)
        + ba_ref[...], 0.0)
    y = (jnp.dot(z, wb_ref[...], preferred_element_type=jnp.float32)
         + bb_ref[...])
    y_ref[...] = y

    @pl.when(i == 0)
    def _():
        sum_ref[...] = jnp.zeros_like(sum_ref)
        sq_ref[...] = jnp.zeros_like(sq_ref)

    sum_ref[...] += jnp.sum(y, axis=0, keepdims=True)
    sq_ref[...] += jnp.sum(y * y, axis=0, keepdims=True)


def _mlp_stats(h, parts, wa, ba, wb, bb, blk):
    n, d = h.shape
    hh = wb.shape[1]
    nblk = n // blk
    return pl.pallas_call(
        functools.partial(_mlp_body, n),
        grid=(nblk,),
        in_specs=[
            pl.BlockSpec((blk, d), lambda i: (i, 0)),
            pl.BlockSpec((NC, blk, d), lambda i: (0, i, 0)),
            pl.BlockSpec((d, hh), lambda i: (0, 0)),
            pl.BlockSpec((1, hh), lambda i: (0, 0)),
            pl.BlockSpec((hh, hh), lambda i: (0, 0)),
            pl.BlockSpec((1, hh), lambda i: (0, 0)),
        ],
        out_specs=[
            pl.BlockSpec((blk, hh), lambda i: (i, 0)),
            pl.BlockSpec((1, hh), lambda i: (0, 0)),
            pl.BlockSpec((1, hh), lambda i: (0, 0)),
        ],
        out_shape=[
            jax.ShapeDtypeStruct((n, hh), jnp.float32),
            jax.ShapeDtypeStruct((1, hh), jnp.float32),
            jax.ShapeDtypeStruct((1, hh), jnp.float32),
        ],
    )(h, parts, wa, ba, wb, bb)


def _bn_body(n, y_ref, sum_ref, sq_ref, g_ref, be_ref, o_ref):
    mean = sum_ref[...] / n
    var = sq_ref[...] / n - mean * mean
    scale = lax.rsqrt(var + 1e-5) * g_ref[...]
    o_ref[...] = jnp.maximum((y_ref[...] - mean) * scale + be_ref[...], 0.0)


def _bn_relu(y, s, q, g, be, blk):
    n, hh = y.shape
    return pl.pallas_call(
        functools.partial(_bn_body, n),
        grid=(n // blk,),
        in_specs=[
            pl.BlockSpec((blk, hh), lambda i: (i, 0)),
            pl.BlockSpec((1, hh), lambda i: (0, 0)),
            pl.BlockSpec((1, hh), lambda i: (0, 0)),
            pl.BlockSpec((1, hh), lambda i: (0, 0)),
            pl.BlockSpec((1, hh), lambda i: (0, 0)),
        ],
        out_specs=pl.BlockSpec((blk, hh), lambda i: (i, 0)),
        out_shape=jax.ShapeDtypeStruct((n, hh), jnp.float32),
    )(y, s, q, g, be)


def _pool_body(n, g, nblk, y_ref, sum_ref, sq_ref, g2_ref, be_ref, b_ref,
               w5_ref, b5_ref, o_ref, acc_ref, cnt_ref):
    i = pl.program_id(0)

    @pl.when(i == 0)
    def _():
        acc_ref[...] = jnp.zeros_like(acc_ref)
        cnt_ref[...] = jnp.zeros_like(cnt_ref)

    mean = sum_ref[...] / n
    var = sq_ref[...] / n - mean * mean
    scale = lax.rsqrt(var + 1e-5) * g2_ref[...]
    h = jnp.maximum((y_ref[...] - mean) * scale + be_ref[...], 0.0)

    mask = (b_ref[...] == lax.broadcasted_iota(jnp.int32, (1, g), 1)
            ).astype(jnp.float32)
    acc_ref[...] += lax.dot_general(mask, h, (((0,), (0,)), ((), ())))
    cnt_ref[...] += jnp.sum(mask, axis=0, keepdims=True)

    @pl.when(i == nblk - 1)
    def _():
        pooled = acc_ref[...] / jnp.maximum(cnt_ref[...], 1.0).reshape(g, 1)
        logits = (jnp.dot(pooled, w5_ref[...],
                          preferred_element_type=jnp.float32) + b5_ref[...])
        m = jnp.max(logits, axis=1, keepdims=True)
        lse = jnp.log(jnp.sum(jnp.exp(logits - m), axis=1, keepdims=True)) + m
        o_ref[...] = logits - lse


def _pool_classify(y, s, q, g2, be2, batch2d, w5p, b5p, g, blk):
    n, hh = y.shape
    cp = w5p.shape[1]
    nblk = n // blk
    return pl.pallas_call(
        functools.partial(_pool_body, n, g, nblk),
        grid=(nblk,),
        in_specs=[
            pl.BlockSpec((blk, hh), lambda i: (i, 0)),
            pl.BlockSpec((1, hh), lambda i: (0, 0)),
            pl.BlockSpec((1, hh), lambda i: (0, 0)),
            pl.BlockSpec((1, hh), lambda i: (0, 0)),
            pl.BlockSpec((1, hh), lambda i: (0, 0)),
            pl.BlockSpec((blk, 1), lambda i: (i, 0)),
            pl.BlockSpec((hh, cp), lambda i: (0, 0)),
            pl.BlockSpec((1, cp), lambda i: (0, 0)),
        ],
        out_specs=pl.BlockSpec((g, cp), lambda i: (0, 0)),
        out_shape=jax.ShapeDtypeStruct((g, cp), jnp.float32),
        scratch_shapes=[
            pltpu.VMEM((g, hh), jnp.float32),
            pltpu.VMEM((1, g), jnp.float32),
        ],
    )(y, s, q, g2, be2, batch2d, w5p, b5p)


def kernel(x, edge_index, batch, W1, b1, W2, b2, g1, be1, W3, b3, W4, b4,
           g2, be2, W5, b5):
    n, d = x.shape
    e = edge_index.shape[1]
    g = 64
    blk = 2000

    nch = -(-e // K)
    nch8 = -(-nch // 8) * 8
    if e != nch8 * K:
        pad = nch8 * K - e
        edge_index = jnp.concatenate(
            [edge_index,
             jnp.stack([jnp.zeros((pad,), jnp.int32),
                        jnp.full((pad,), n, jnp.int32)])], axis=1)
    edge3d = edge_index.reshape(2, nch8, K)
    t0c = min(nch - 1, max(1, int(round(FRAC0 * nch))))
    cb0, cr0 = divmod(t0c, NS)
    cb1, cr1 = divmod(nch - t0c, NS)
    cmax8 = -(-(max(cb0, cb1) + 1 + 7) // 8) * 8
    npad = ((n + NS * 8 - 1) // (NS * 8)) * (NS * 8)
    if npad == n:
        npad += NS * 8
    zblk = jnp.zeros((npad // NS, d), jnp.float32)

    batch2d = batch.reshape(n, 1)
    cpad = 128
    c = W5.shape[1]
    w5p = jnp.pad(W5, ((0, 0), (0, cpad - c)))
    b5p = jnp.pad(b5, (0, cpad - c), constant_values=-1e30).reshape(1, cpad)

    p1 = _sc_agg(x, edge3d, zblk, npad, nch8, cmax8, t0c, cb0, cr0, cb1, cr1)
    y1, s1, q1 = _mlp_stats(x, p1, W1, b1.reshape(1, -1),
                            W2, b2.reshape(1, -1), blk)
    h1 = _bn_relu(y1, s1, q1, g1.reshape(1, -1), be1.reshape(1, -1), blk)

    p2 = _sc_agg(h1, edge3d, zblk, npad, nch8, cmax8, t0c, cb0, cr0, cb1, cr1)
    y2, s2, q2 = _mlp_stats(h1, p2, W3, b3.reshape(1, -1),
                            W4, b4.reshape(1, -1), blk)

    out = _pool_classify(y2, s2, q2, g2.reshape(1, -1), be2.reshape(1, -1),
                         batch2d, w5p, b5p, g, blk)
    return out[:, :c]


# speedup vs baseline: 1.0550x; 1.0200x over previous
import functools

import numpy as np
import jax
import jax.numpy as jnp
from jax import lax
from jax.experimental import pallas as pl
from jax.experimental.pallas import tpu as pltpu
from jax.experimental.pallas import tpu_sc as plsc

NC = 2
NS = 16
NW = NC * NS
K = 128
FRAC0 = 0.548


def _sc_agg(h, edge3d, zblk, npad, nch8, cmax8, t0c, b0, r0, b1, r1):
    n, d = h.shape
    rpt = npad // NS

    mesh = plsc.VectorSubcoreMesh(
        core_axis_name="c", subcore_axis_name="s",
        num_cores=NC, num_subcores=NS)

    @functools.partial(
        pl.kernel,
        out_type=jax.ShapeDtypeStruct((NC, npad, d), jnp.float32),
        mesh=mesh,
        scratch_types=[
            pltpu.VMEM((cmax8, K), jnp.int32),
            pltpu.VMEM((cmax8, K), jnp.int32),
            pltpu.VMEM((K, d), jnp.float32),
            pltpu.VMEM_SHARED((npad, d), jnp.float32),
            pltpu.SemaphoreType.DMA,
        ],
    )
    def agg(h_hbm, e_hbm, z_hbm, out_hbm, src_v, dst_v, rows_v, acc, sem):
        c = lax.axis_index("c")
        s = lax.axis_index("s")
        pltpu.sync_copy(z_hbm, acc.at[pl.ds(s * rpt, rpt)])
        b = jnp.where(c == 0, b0, b1)
        r = jnp.where(c == 0, r0, r1)
        cw = b + (s < r).astype(jnp.int32)
        off = (jnp.where(c == 0, 0, t0c) + s * b
               + jnp.minimum(s, r))
        astart = jnp.minimum((off // 8) * 8, nch8 - cmax8)
        rem = off - astart
        pltpu.sync_copy(e_hbm.at[0].at[pl.ds(astart, cmax8)], src_v)
        pltpu.sync_copy(e_hbm.at[1].at[pl.ds(astart, cmax8)], dst_v)
        plsc.subcore_barrier()

        def chunk(j, carry):
            pltpu.async_copy(h_hbm.at[src_v.at[j + rem]], rows_v, sem).wait()
            pltpu.sync_copy(rows_v, acc.at[dst_v.at[j + rem]], add=True)
            return carry

        lax.fori_loop(0, cw, chunk, 0, unroll=False)
        plsc.subcore_barrier()
        pltpu.sync_copy(acc.at[pl.ds(s * rpt, rpt)],
                        out_hbm.at[c].at[pl.ds(s * rpt, rpt)])

    return agg(h, edge3d, zblk)


def _mlp_body(n, h_ref, p_ref, wa_ref, ba_ref, wb_ref, bb_ref,
              y_ref, sum_ref, sq_ref):
    i = pl.program_id(0)
    z = h_ref[...] + p_ref[0] + p_ref[1]
    z = jnp.maximum(
        jnp.dot(z, wa_ref[...], preferred_element_type=jnp.float32)
        + ba_ref[...], 0.0)
    y = (jnp.dot(z, wb_ref[...], preferred_element_type=jnp.float32)
         + bb_ref[...])
    y_ref[...] = y

    @pl.when(i == 0)
    def _():
        sum_ref[...] = jnp.zeros_like(sum_ref)
        sq_ref[...] = jnp.zeros_like(sq_ref)

    sum_ref[...] += jnp.sum(y, axis=0, keepdims=True)
    sq_ref[...] += jnp.sum(y * y, axis=0, keepdims=True)


def _mlp_stats(h, parts, wa, ba, wb, bb, blk):
    n, d = h.shape
    hh = wb.shape[1]
    nblk = n // blk
    return pl.pallas_call(
        functools.partial(_mlp_body, n),
        grid=(nblk,),
        in_specs=[
            pl.BlockSpec((blk, d), lambda i: (i, 0)),
            pl.BlockSpec((NC, blk, d), lambda i: (0, i, 0)),
            pl.BlockSpec((d, hh), lambda i: (0, 0)),
            pl.BlockSpec((1, hh), lambda i: (0, 0)),
            pl.BlockSpec((hh, hh), lambda i: (0, 0)),
            pl.BlockSpec((1, hh), lambda i: (0, 0)),
        ],
        out_specs=[
            pl.BlockSpec((blk, hh), lambda i: (i, 0)),
            pl.BlockSpec((1, hh), lambda i: (0, 0)),
            pl.BlockSpec((1, hh), lambda i: (0, 0)),
        ],
        out_shape=[
            jax.ShapeDtypeStruct((n, hh), jnp.float32),
            jax.ShapeDtypeStruct((1, hh), jnp.float32),
            jax.ShapeDtypeStruct((1, hh), jnp.float32),
        ],
    )(h, parts, wa, ba, wb, bb)


def _bn_body(n, y_ref, sum_ref, sq_ref, g_ref, be_ref, o_ref):
    mean = sum_ref[...] / n
    var = sq_ref[...] / n - mean * mean
    scale = lax.rsqrt(var + 1e-5) * g_ref[...]
    o_ref[...] = jnp.maximum((y_ref[...] - mean) * scale + be_ref[...], 0.0)


def _bn_relu(y, s, q, g, be, blk):
    n, hh = y.shape
    return pl.pallas_call(
        functools.partial(_bn_body, n),
        grid=(n // blk,),
        in_specs=[
            pl.BlockSpec((blk, hh), lambda i: (i, 0)),
            pl.BlockSpec((1, hh), lambda i: (0, 0)),
            pl.BlockSpec((1, hh), lambda i: (0, 0)),
            pl.BlockSpec((1, hh), lambda i: (0, 0)),
            pl.BlockSpec((1, hh), lambda i: (0, 0)),
        ],
        out_specs=pl.BlockSpec((blk, hh), lambda i: (i, 0)),
        out_shape=jax.ShapeDtypeStruct((n, hh), jnp.float32),
    )(y, s, q, g, be)


def _pool_body(n, g, nblk, y_ref, sum_ref, sq_ref, g2_ref, be_ref, b_ref,
               w5_ref, b5_ref, o_ref, acc_ref, cnt_ref):
    i = pl.program_id(0)

    @pl.when(i == 0)
    def _():
        acc_ref[...] = jnp.zeros_like(acc_ref)
        cnt_ref[...] = jnp.zeros_like(cnt_ref)

    mean = sum_ref[...] / n
    var = sq_ref[...] / n - mean * mean
    scale = lax.rsqrt(var + 1e-5) * g2_ref[...]
    h = jnp.maximum((y_ref[...] - mean) * scale + be_ref[...], 0.0)

    mask = (b_ref[...] == lax.broadcasted_iota(jnp.int32, (1, g), 1)
            ).astype(jnp.float32)
    acc_ref[...] += lax.dot_general(mask, h, (((0,), (0,)), ((), ())))
    cnt_ref[...] += jnp.sum(mask, axis=0, keepdims=True)

    @pl.when(i == nblk - 1)
    def _():
        pooled = acc_ref[...] / jnp.maximum(cnt_ref[...], 1.0).reshape(g, 1)
        logits = (jnp.dot(pooled, w5_ref[...],
                          preferred_element_type=jnp.float32) + b5_ref[...])
        m = jnp.max(logits, axis=1, keepdims=True)
        lse = jnp.log(jnp.sum(jnp.exp(logits - m), axis=1, keepdims=True)) + m
        o_ref[...] = logits - lse


def _pool_classify(y, s, q, g2, be2, batch2d, w5p, b5p, g, blk):
    n, hh = y.shape
    cp = w5p.shape[1]
    nblk = n // blk
    return pl.pallas_call(
        functools.partial(_pool_body, n, g, nblk),
        grid=(nblk,),
        in_specs=[
            pl.BlockSpec((blk, hh), lambda i: (i, 0)),
            pl.BlockSpec((1, hh), lambda i: (0, 0)),
            pl.BlockSpec((1, hh), lambda i: (0, 0)),
            pl.BlockSpec((1, hh), lambda i: (0, 0)),
            pl.BlockSpec((1, hh), lambda i: (0, 0)),
            pl.BlockSpec((blk, 1), lambda i: (i, 0)),
            pl.BlockSpec((hh, cp), lambda i: (0, 0)),
            pl.BlockSpec((1, cp), lambda i: (0, 0)),
        ],
        out_specs=pl.BlockSpec((g, cp), lambda i: (0, 0)),
        out_shape=jax.ShapeDtypeStruct((g, cp), jnp.float32),
        scratch_shapes=[
            pltpu.VMEM((g, hh), jnp.float32),
            pltpu.VMEM((1, g), jnp.float32),
        ],
    )(y, s, q, g2, be2, batch2d, w5p, b5p)


def kernel(x, edge_index, batch, W1, b1, W2, b2, g1, be1, W3, b3, W4, b4,
           g2, be2, W5, b5):
    n, d = x.shape
    e = edge_index.shape[1]
    g = 64
    blk = 2000

    nch = -(-e // K)
    nch8 = -(-nch // 8) * 8
    if e != nch8 * K:
        pad = nch8 * K - e
        edge_index = jnp.concatenate(
            [edge_index,
             jnp.stack([jnp.zeros((pad,), jnp.int32),
                        jnp.full((pad,), n, jnp.int32)])], axis=1)
    edge3d = edge_index.reshape(2, nch8, K)
    t0c = min(nch - 1, max(1, int(round(FRAC0 * nch))))
    cb0, cr0 = divmod(t0c, NS)
    cb1, cr1 = divmod(nch - t0c, NS)
    cmax8 = -(-(max(cb0, cb1) + 1 + 7) // 8) * 8
    npad = ((n + NS * 8 - 1) // (NS * 8)) * (NS * 8)
    if npad == n:
        npad += NS * 8
    zblk = jnp.zeros((npad // NS, d), jnp.float32)

    batch2d = batch.reshape(n, 1)
    cpad = 128
    c = W5.shape[1]
    w5p = jnp.pad(W5, ((0, 0), (0, cpad - c)))
    b5p = jnp.pad(b5, (0, cpad - c), constant_values=-1e30).reshape(1, cpad)

    p1 = _sc_agg(x, edge3d, zblk, npad, nch8, cmax8, t0c, cb0, cr0, cb1, cr1)
    y1, s1, q1 = _mlp_stats(x, p1, W1, b1.reshape(1, -1),
                            W2, b2.reshape(1, -1), blk)
    h1 = _bn_relu(y1, s1, q1, g1.reshape(1, -1), be1.reshape(1, -1), blk)

    p2 = _sc_agg(h1, edge3d, zblk, npad, nch8, cmax8, t0c, cb0, cr0, cb1, cr1)
    y2, s2, q2 = _mlp_stats(h1, p2, W3, b3.reshape(1, -1),
                            W4, b4.reshape(1, -1), blk)

    out = _pool_classify(y2, s2, q2, g2.reshape(1, -1), be2.reshape(1, -1),
                         batch2d, w5p, b5p, g, blk)
    return out[:, :c]


# speedup vs baseline: 1.0731x; 1.0172x over previous
import functools

import numpy as np
import jax
import jax.numpy as jnp
from jax import lax
from jax.experimental import pallas as pl
from jax.experimental.pallas import tpu as pltpu
from jax.experimental.pallas import tpu_sc as plsc

NC = 2
NS = 16
NW = NC * NS
K = 128
FRAC0 = 0.536


def _sc_agg(h, edge3d, zblk, npad, nch8, cmax8, t0c, b0, r0, b1, r1):
    n, d = h.shape
    rpt = npad // NS

    mesh = plsc.VectorSubcoreMesh(
        core_axis_name="c", subcore_axis_name="s",
        num_cores=NC, num_subcores=NS)

    @functools.partial(
        pl.kernel,
        out_type=jax.ShapeDtypeStruct((NC, npad, d), jnp.float32),
        mesh=mesh,
        scratch_types=[
            pltpu.VMEM((cmax8, K), jnp.int32),
            pltpu.VMEM((cmax8, K), jnp.int32),
            pltpu.VMEM((K, d), jnp.float32),
            pltpu.VMEM_SHARED((npad, d), jnp.float32),
            pltpu.SemaphoreType.DMA,
        ],
    )
    def agg(h_hbm, e_hbm, z_hbm, out_hbm, src_v, dst_v, rows_v, acc, sem):
        c = lax.axis_index("c")
        s = lax.axis_index("s")
        pltpu.sync_copy(z_hbm, acc.at[pl.ds(s * rpt, rpt)])
        b = jnp.where(c == 0, b0, b1)
        r = jnp.where(c == 0, r0, r1)
        cw = b + (s < r).astype(jnp.int32)
        off = (jnp.where(c == 0, 0, t0c) + s * b
               + jnp.minimum(s, r))
        astart = jnp.minimum((off // 8) * 8, nch8 - cmax8)
        rem = off - astart
        pltpu.sync_copy(e_hbm.at[0].at[pl.ds(astart, cmax8)], src_v)
        pltpu.sync_copy(e_hbm.at[1].at[pl.ds(astart, cmax8)], dst_v)
        plsc.subcore_barrier()

        def chunk(j, carry):
            pltpu.async_copy(h_hbm.at[src_v.at[j + rem]], rows_v, sem).wait()
            pltpu.sync_copy(rows_v, acc.at[dst_v.at[j + rem]], add=True)
            return carry

        lax.fori_loop(0, cw, chunk, 0, unroll=False)
        plsc.subcore_barrier()
        pltpu.sync_copy(acc.at[pl.ds(s * rpt, rpt)],
                        out_hbm.at[c].at[pl.ds(s * rpt, rpt)])

    return agg(h, edge3d, zblk)


def _mlp_body(n, h_ref, p_ref, wa_ref, ba_ref, wb_ref, bb_ref,
              y_ref, sum_ref, sq_ref):
    i = pl.program_id(0)
    z = h_ref[...] + p_ref[0] + p_ref[1]
    z = jnp.maximum(
        jnp.dot(z, wa_ref[...], preferred_element_type=jnp.float32)
        + ba_ref[...], 0.0)
    y = (jnp.dot(z, wb_ref[...], preferred_element_type=jnp.float32)
         + bb_ref[...])
    y_ref[...] = y

    @pl.when(i == 0)
    def _():
        sum_ref[...] = jnp.zeros_like(sum_ref)
        sq_ref[...] = jnp.zeros_like(sq_ref)

    sum_ref[...] += jnp.sum(y, axis=0, keepdims=True)
    sq_ref[...] += jnp.sum(y * y, axis=0, keepdims=True)


def _mlp_stats(h, parts, wa, ba, wb, bb, blk):
    n, d = h.shape
    hh = wb.shape[1]
    nblk = n // blk
    return pl.pallas_call(
        functools.partial(_mlp_body, n),
        grid=(nblk,),
        in_specs=[
            pl.BlockSpec((blk, d), lambda i: (i, 0)),
            pl.BlockSpec((NC, blk, d), lambda i: (0, i, 0)),
            pl.BlockSpec((d, hh), lambda i: (0, 0)),
            pl.BlockSpec((1, hh), lambda i: (0, 0)),
            pl.BlockSpec((hh, hh), lambda i: (0, 0)),
            pl.BlockSpec((1, hh), lambda i: (0, 0)),
        ],
        out_specs=[
            pl.BlockSpec((blk, hh), lambda i: (i, 0)),
            pl.BlockSpec((1, hh), lambda i: (0, 0)),
            pl.BlockSpec((1, hh), lambda i: (0, 0)),
        ],
        out_shape=[
            jax.ShapeDtypeStruct((n, hh), jnp.float32),
            jax.ShapeDtypeStruct((1, hh), jnp.float32),
            jax.ShapeDtypeStruct((1, hh), jnp.float32),
        ],
    )(h, parts, wa, ba, wb, bb)


def _bn_body(n, y_ref, sum_ref, sq_ref, g_ref, be_ref, o_ref):
    mean = sum_ref[...] / n
    var = sq_ref[...] / n - mean * mean
    scale = lax.rsqrt(var + 1e-5) * g_ref[...]
    o_ref[...] = jnp.maximum((y_ref[...] - mean) * scale + be_ref[...], 0.0)


def _bn_relu(y, s, q, g, be, blk):
    n, hh = y.shape
    return pl.pallas_call(
        functools.partial(_bn_body, n),
        grid=(n // blk,),
        in_specs=[
            pl.BlockSpec((blk, hh), lambda i: (i, 0)),
            pl.BlockSpec((1, hh), lambda i: (0, 0)),
            pl.BlockSpec((1, hh), lambda i: (0, 0)),
            pl.BlockSpec((1, hh), lambda i: (0, 0)),
            pl.BlockSpec((1, hh), lambda i: (0, 0)),
        ],
        out_specs=pl.BlockSpec((blk, hh), lambda i: (i, 0)),
        out_shape=jax.ShapeDtypeStruct((n, hh), jnp.float32),
    )(y, s, q, g, be)


def _pool_body(n, g, nblk, y_ref, sum_ref, sq_ref, g2_ref, be_ref, b_ref,
               w5_ref, b5_ref, o_ref, acc_ref, cnt_ref):
    i = pl.program_id(0)

    @pl.when(i == 0)
    def _():
        acc_ref[...] = jnp.zeros_like(acc_ref)
        cnt_ref[...] = jnp.zeros_like(cnt_ref)

    mean = sum_ref[...] / n
    var = sq_ref[...] / n - mean * mean
    scale = lax.rsqrt(var + 1e-5) * g2_ref[...]
    h = jnp.maximum((y_ref[...] - mean) * scale + be_ref[...], 0.0)

    mask = (b_ref[...] == lax.broadcasted_iota(jnp.int32, (1, g), 1)
            ).astype(jnp.float32)
    acc_ref[...] += lax.dot_general(mask, h, (((0,), (0,)), ((), ())))
    cnt_ref[...] += jnp.sum(mask, axis=0, keepdims=True)

    @pl.when(i == nblk - 1)
    def _():
        pooled = acc_ref[...] / jnp.maximum(cnt_ref[...], 1.0).reshape(g, 1)
        logits = (jnp.dot(pooled, w5_ref[...],
                          preferred_element_type=jnp.float32) + b5_ref[...])
        m = jnp.max(logits, axis=1, keepdims=True)
        lse = jnp.log(jnp.sum(jnp.exp(logits - m), axis=1, keepdims=True)) + m
        o_ref[...] = logits - lse


def _pool_classify(y, s, q, g2, be2, batch2d, w5p, b5p, g, blk):
    n, hh = y.shape
    cp = w5p.shape[1]
    nblk = n // blk
    return pl.pallas_call(
        functools.partial(_pool_body, n, g, nblk),
        grid=(nblk,),
        in_specs=[
            pl.BlockSpec((blk, hh), lambda i: (i, 0)),
            pl.BlockSpec((1, hh), lambda i: (0, 0)),
            pl.BlockSpec((1, hh), lambda i: (0, 0)),
            pl.BlockSpec((1, hh), lambda i: (0, 0)),
            pl.BlockSpec((1, hh), lambda i: (0, 0)),
            pl.BlockSpec((blk, 1), lambda i: (i, 0)),
            pl.BlockSpec((hh, cp), lambda i: (0, 0)),
            pl.BlockSpec((1, cp), lambda i: (0, 0)),
        ],
        out_specs=pl.BlockSpec((g, cp), lambda i: (0, 0)),
        out_shape=jax.ShapeDtypeStruct((g, cp), jnp.float32),
        scratch_shapes=[
            pltpu.VMEM((g, hh), jnp.float32),
            pltpu.VMEM((1, g), jnp.float32),
        ],
    )(y, s, q, g2, be2, batch2d, w5p, b5p)


def kernel(x, edge_index, batch, W1, b1, W2, b2, g1, be1, W3, b3, W4, b4,
           g2, be2, W5, b5):
    n, d = x.shape
    e = edge_index.shape[1]
    g = 64
    blk = 2000

    nch = -(-e // K)
    nch8 = -(-nch // 8) * 8
    if e != nch8 * K:
        pad = nch8 * K - e
        edge_index = jnp.concatenate(
            [edge_index,
             jnp.stack([jnp.zeros((pad,), jnp.int32),
                        jnp.full((pad,), n, jnp.int32)])], axis=1)
    edge3d = edge_index.reshape(2, nch8, K)
    t0c = min(nch - 1, max(1, int(round(FRAC0 * nch))))
    cb0, cr0 = divmod(t0c, NS)
    cb1, cr1 = divmod(nch - t0c, NS)
    cmax8 = -(-(max(cb0, cb1) + 1 + 7) // 8) * 8
    npad = ((n + NS * 8 - 1) // (NS * 8)) * (NS * 8)
    if npad == n:
        npad += NS * 8
    zblk = jnp.zeros((npad // NS, d), jnp.float32)

    batch2d = batch.reshape(n, 1)
    cpad = 128
    c = W5.shape[1]
    w5p = jnp.pad(W5, ((0, 0), (0, cpad - c)))
    b5p = jnp.pad(b5, (0, cpad - c), constant_values=-1e30).reshape(1, cpad)

    p1 = _sc_agg(x, edge3d, zblk, npad, nch8, cmax8, t0c, cb0, cr0, cb1, cr1)
    y1, s1, q1 = _mlp_stats(x, p1, W1, b1.reshape(1, -1),
                            W2, b2.reshape(1, -1), blk)
    h1 = _bn_relu(y1, s1, q1, g1.reshape(1, -1), be1.reshape(1, -1), blk)

    p2 = _sc_agg(h1, edge3d, zblk, npad, nch8, cmax8, t0c, cb0, cr0, cb1, cr1)
    y2, s2, q2 = _mlp_stats(h1, p2, W3, b3.reshape(1, -1),
                            W4, b4.reshape(1, -1), blk)

    out = _pool_classify(y2, s2, q2, g2.reshape(1, -1), be2.reshape(1, -1),
                         batch2d, w5p, b5p, g, blk)
    return out[:, :c]


# speedup vs baseline: 1.0926x; 1.0181x over previous
import functools

import numpy as np
import jax
import jax.numpy as jnp
from jax import lax
from jax.experimental import pallas as pl
from jax.experimental.pallas import tpu as pltpu
from jax.experimental.pallas import tpu_sc as plsc

NC = 2
NS = 16
NW = NC * NS
K = 128
FRAC0 = 0.520


def _sc_agg(h, edge3d, zblk, npad, nch8, cmax8, t0c, b0, r0, b1, r1):
    n, d = h.shape
    rpt = npad // NS

    mesh = plsc.VectorSubcoreMesh(
        core_axis_name="c", subcore_axis_name="s",
        num_cores=NC, num_subcores=NS)

    @functools.partial(
        pl.kernel,
        out_type=jax.ShapeDtypeStruct((NC, npad, d), jnp.float32),
        mesh=mesh,
        scratch_types=[
            pltpu.VMEM((cmax8, K), jnp.int32),
            pltpu.VMEM((cmax8, K), jnp.int32),
            pltpu.VMEM((K, d), jnp.float32),
            pltpu.VMEM_SHARED((npad, d), jnp.float32),
            pltpu.SemaphoreType.DMA,
        ],
    )
    def agg(h_hbm, e_hbm, z_hbm, out_hbm, src_v, dst_v, rows_v, acc, sem):
        c = lax.axis_index("c")
        s = lax.axis_index("s")
        pltpu.sync_copy(z_hbm, acc.at[pl.ds(s * rpt, rpt)])
        b = jnp.where(c == 0, b0, b1)
        r = jnp.where(c == 0, r0, r1)
        cw = b + (s < r).astype(jnp.int32)
        off = (jnp.where(c == 0, 0, t0c) + s * b
               + jnp.minimum(s, r))
        astart = jnp.minimum((off // 8) * 8, nch8 - cmax8)
        rem = off - astart
        pltpu.sync_copy(e_hbm.at[0].at[pl.ds(astart, cmax8)], src_v)
        pltpu.sync_copy(e_hbm.at[1].at[pl.ds(astart, cmax8)], dst_v)
        plsc.subcore_barrier()

        def chunk(j, carry):
            pltpu.async_copy(h_hbm.at[src_v.at[j + rem]], rows_v, sem).wait()
            pltpu.sync_copy(rows_v, acc.at[dst_v.at[j + rem]], add=True)
            return carry

        lax.fori_loop(0, cw, chunk, 0, unroll=False)
        plsc.subcore_barrier()
        pltpu.sync_copy(acc.at[pl.ds(s * rpt, rpt)],
                        out_hbm.at[c].at[pl.ds(s * rpt, rpt)])

    return agg(h, edge3d, zblk)


def _mlp_body(n, h_ref, p_ref, wa_ref, ba_ref, wb_ref, bb_ref,
              y_ref, sum_ref, sq_ref):
    i = pl.program_id(0)
    z = h_ref[...] + p_ref[0] + p_ref[1]
    z = jnp.maximum(
        jnp.dot(z, wa_ref[...], preferred_element_type=jnp.float32)
        + ba_ref[...], 0.0)
    y = (jnp.dot(z, wb_ref[...], preferred_element_type=jnp.float32)
         + bb_ref[...])
    y_ref[...] = y

    @pl.when(i == 0)
    def _():
        sum_ref[...] = jnp.zeros_like(sum_ref)
        sq_ref[...] = jnp.zeros_like(sq_ref)

    sum_ref[...] += jnp.sum(y, axis=0, keepdims=True)
    sq_ref[...] += jnp.sum(y * y, axis=0, keepdims=True)


def _mlp_stats(h, parts, wa, ba, wb, bb, blk):
    n, d = h.shape
    hh = wb.shape[1]
    nblk = n // blk
    return pl.pallas_call(
        functools.partial(_mlp_body, n),
        grid=(nblk,),
        in_specs=[
            pl.BlockSpec((blk, d), lambda i: (i, 0)),
            pl.BlockSpec((NC, blk, d), lambda i: (0, i, 0)),
            pl.BlockSpec((d, hh), lambda i: (0, 0)),
            pl.BlockSpec((1, hh), lambda i: (0, 0)),
            pl.BlockSpec((hh, hh), lambda i: (0, 0)),
            pl.BlockSpec((1, hh), lambda i: (0, 0)),
        ],
        out_specs=[
            pl.BlockSpec((blk, hh), lambda i: (i, 0)),
            pl.BlockSpec((1, hh), lambda i: (0, 0)),
            pl.BlockSpec((1, hh), lambda i: (0, 0)),
        ],
        out_shape=[
            jax.ShapeDtypeStruct((n, hh), jnp.float32),
            jax.ShapeDtypeStruct((1, hh), jnp.float32),
            jax.ShapeDtypeStruct((1, hh), jnp.float32),
        ],
    )(h, parts, wa, ba, wb, bb)


def _bn_body(n, y_ref, sum_ref, sq_ref, g_ref, be_ref, o_ref):
    mean = sum_ref[...] / n
    var = sq_ref[...] / n - mean * mean
    scale = lax.rsqrt(var + 1e-5) * g_ref[...]
    o_ref[...] = jnp.maximum((y_ref[...] - mean) * scale + be_ref[...], 0.0)


def _bn_relu(y, s, q, g, be, blk):
    n, hh = y.shape
    return pl.pallas_call(
        functools.partial(_bn_body, n),
        grid=(n // blk,),
        in_specs=[
            pl.BlockSpec((blk, hh), lambda i: (i, 0)),
            pl.BlockSpec((1, hh), lambda i: (0, 0)),
            pl.BlockSpec((1, hh), lambda i: (0, 0)),
            pl.BlockSpec((1, hh), lambda i: (0, 0)),
            pl.BlockSpec((1, hh), lambda i: (0, 0)),
        ],
        out_specs=pl.BlockSpec((blk, hh), lambda i: (i, 0)),
        out_shape=jax.ShapeDtypeStruct((n, hh), jnp.float32),
    )(y, s, q, g, be)


def _pool_body(n, g, nblk, y_ref, sum_ref, sq_ref, g2_ref, be_ref, b_ref,
               w5_ref, b5_ref, o_ref, acc_ref, cnt_ref):
    i = pl.program_id(0)

    @pl.when(i == 0)
    def _():
        acc_ref[...] = jnp.zeros_like(acc_ref)
        cnt_ref[...] = jnp.zeros_like(cnt_ref)

    mean = sum_ref[...] / n
    var = sq_ref[...] / n - mean * mean
    scale = lax.rsqrt(var + 1e-5) * g2_ref[...]
    h = jnp.maximum((y_ref[...] - mean) * scale + be_ref[...], 0.0)

    mask = (b_ref[...] == lax.broadcasted_iota(jnp.int32, (1, g), 1)
            ).astype(jnp.float32)
    acc_ref[...] += lax.dot_general(mask, h, (((0,), (0,)), ((), ())))
    cnt_ref[...] += jnp.sum(mask, axis=0, keepdims=True)

    @pl.when(i == nblk - 1)
    def _():
        pooled = acc_ref[...] / jnp.maximum(cnt_ref[...], 1.0).reshape(g, 1)
        logits = (jnp.dot(pooled, w5_ref[...],
                          preferred_element_type=jnp.float32) + b5_ref[...])
        m = jnp.max(logits, axis=1, keepdims=True)
        lse = jnp.log(jnp.sum(jnp.exp(logits - m), axis=1, keepdims=True)) + m
        o_ref[...] = logits - lse


def _pool_classify(y, s, q, g2, be2, batch2d, w5p, b5p, g, blk):
    n, hh = y.shape
    cp = w5p.shape[1]
    nblk = n // blk
    return pl.pallas_call(
        functools.partial(_pool_body, n, g, nblk),
        grid=(nblk,),
        in_specs=[
            pl.BlockSpec((blk, hh), lambda i: (i, 0)),
            pl.BlockSpec((1, hh), lambda i: (0, 0)),
            pl.BlockSpec((1, hh), lambda i: (0, 0)),
            pl.BlockSpec((1, hh), lambda i: (0, 0)),
            pl.BlockSpec((1, hh), lambda i: (0, 0)),
            pl.BlockSpec((blk, 1), lambda i: (i, 0)),
            pl.BlockSpec((hh, cp), lambda i: (0, 0)),
            pl.BlockSpec((1, cp), lambda i: (0, 0)),
        ],
        out_specs=pl.BlockSpec((g, cp), lambda i: (0, 0)),
        out_shape=jax.ShapeDtypeStruct((g, cp), jnp.float32),
        scratch_shapes=[
            pltpu.VMEM((g, hh), jnp.float32),
            pltpu.VMEM((1, g), jnp.float32),
        ],
    )(y, s, q, g2, be2, batch2d, w5p, b5p)


def kernel(x, edge_index, batch, W1, b1, W2, b2, g1, be1, W3, b3, W4, b4,
           g2, be2, W5, b5):
    n, d = x.shape
    e = edge_index.shape[1]
    g = 64
    blk = 2000

    nch = -(-e // K)
    nch8 = -(-nch // 8) * 8
    if e != nch8 * K:
        pad = nch8 * K - e
        edge_index = jnp.concatenate(
            [edge_index,
             jnp.stack([jnp.zeros((pad,), jnp.int32),
                        jnp.full((pad,), n, jnp.int32)])], axis=1)
    edge3d = edge_index.reshape(2, nch8, K)
    t0c = min(nch - 1, max(1, int(round(FRAC0 * nch))))
    cb0, cr0 = divmod(t0c, NS)
    cb1, cr1 = divmod(nch - t0c, NS)
    cmax8 = -(-(max(cb0, cb1) + 1 + 7) // 8) * 8
    npad = ((n + NS * 8 - 1) // (NS * 8)) * (NS * 8)
    if npad == n:
        npad += NS * 8
    zblk = jnp.zeros((npad // NS, d), jnp.float32)

    batch2d = batch.reshape(n, 1)
    cpad = 128
    c = W5.shape[1]
    w5p = jnp.pad(W5, ((0, 0), (0, cpad - c)))
    b5p = jnp.pad(b5, (0, cpad - c), constant_values=-1e30).reshape(1, cpad)

    p1 = _sc_agg(x, edge3d, zblk, npad, nch8, cmax8, t0c, cb0, cr0, cb1, cr1)
    y1, s1, q1 = _mlp_stats(x, p1, W1, b1.reshape(1, -1),
                            W2, b2.reshape(1, -1), blk)
    h1 = _bn_relu(y1, s1, q1, g1.reshape(1, -1), be1.reshape(1, -1), blk)

    p2 = _sc_agg(h1, edge3d, zblk, npad, nch8, cmax8, t0c, cb0, cr0, cb1, cr1)
    y2, s2, q2 = _mlp_stats(h1, p2, W3, b3.reshape(1, -1),
                            W4, b4.reshape(1, -1), blk)

    out = _pool_classify(y2, s2, q2, g2.reshape(1, -1), be2.reshape(1, -1),
                         batch2d, w5p, b5p, g, blk)
    return out[:, :c]


# speedup vs baseline: 1.1245x; 1.0292x over previous
import functools

import numpy as np
import jax
import jax.numpy as jnp
from jax import lax
from jax.experimental import pallas as pl
from jax.experimental.pallas import tpu as pltpu
from jax.experimental.pallas import tpu_sc as plsc

NC = 2
NS = 16
NW = NC * NS
K = 128
FRAC0 = 0.500


def _sc_agg(h, edge3d, zblk, npad, nch8, cmax8, t0c, b0, r0, b1, r1):
    n, d = h.shape
    rpt = npad // NS

    mesh = plsc.VectorSubcoreMesh(
        core_axis_name="c", subcore_axis_name="s",
        num_cores=NC, num_subcores=NS)

    @functools.partial(
        pl.kernel,
        out_type=jax.ShapeDtypeStruct((NC, npad, d), jnp.float32),
        mesh=mesh,
        scratch_types=[
            pltpu.VMEM((cmax8, K), jnp.int32),
            pltpu.VMEM((cmax8, K), jnp.int32),
            pltpu.VMEM((K, d), jnp.float32),
            pltpu.VMEM_SHARED((npad, d), jnp.float32),
            pltpu.SemaphoreType.DMA,
        ],
    )
    def agg(h_hbm, e_hbm, z_hbm, out_hbm, src_v, dst_v, rows_v, acc, sem):
        c = lax.axis_index("c")
        s = lax.axis_index("s")
        pltpu.sync_copy(z_hbm, acc.at[pl.ds(s * rpt, rpt)])
        b = jnp.where(c == 0, b0, b1)
        r = jnp.where(c == 0, r0, r1)
        cw = b + (s < r).astype(jnp.int32)
        off = (jnp.where(c == 0, 0, t0c) + s * b
               + jnp.minimum(s, r))
        astart = jnp.minimum((off // 8) * 8, nch8 - cmax8)
        rem = off - astart
        pltpu.sync_copy(e_hbm.at[0].at[pl.ds(astart, cmax8)], src_v)
        pltpu.sync_copy(e_hbm.at[1].at[pl.ds(astart, cmax8)], dst_v)
        plsc.subcore_barrier()

        def chunk(j, carry):
            pltpu.async_copy(h_hbm.at[src_v.at[j + rem]], rows_v, sem).wait()
            pltpu.sync_copy(rows_v, acc.at[dst_v.at[j + rem]], add=True)
            return carry

        lax.fori_loop(0, cw, chunk, 0, unroll=False)
        plsc.subcore_barrier()
        pltpu.sync_copy(acc.at[pl.ds(s * rpt, rpt)],
                        out_hbm.at[c].at[pl.ds(s * rpt, rpt)])

    return agg(h, edge3d, zblk)


def _mlp_body(n, h_ref, p_ref, wa_ref, ba_ref, wb_ref, bb_ref,
              y_ref, sum_ref, sq_ref):
    i = pl.program_id(0)
    z = h_ref[...] + p_ref[0] + p_ref[1]
    z = jnp.maximum(
        jnp.dot(z, wa_ref[...], preferred_element_type=jnp.float32)
        + ba_ref[...], 0.0)
    y = (jnp.dot(z, wb_ref[...], preferred_element_type=jnp.float32)
         + bb_ref[...])
    y_ref[...] = y

    @pl.when(i == 0)
    def _():
        sum_ref[...] = jnp.zeros_like(sum_ref)
        sq_ref[...] = jnp.zeros_like(sq_ref)

    sum_ref[...] += jnp.sum(y, axis=0, keepdims=True)
    sq_ref[...] += jnp.sum(y * y, axis=0, keepdims=True)


def _mlp_stats(h, parts, wa, ba, wb, bb, blk):
    n, d = h.shape
    hh = wb.shape[1]
    nblk = n // blk
    return pl.pallas_call(
        functools.partial(_mlp_body, n),
        grid=(nblk,),
        in_specs=[
            pl.BlockSpec((blk, d), lambda i: (i, 0)),
            pl.BlockSpec((NC, blk, d), lambda i: (0, i, 0)),
            pl.BlockSpec((d, hh), lambda i: (0, 0)),
            pl.BlockSpec((1, hh), lambda i: (0, 0)),
            pl.BlockSpec((hh, hh), lambda i: (0, 0)),
            pl.BlockSpec((1, hh), lambda i: (0, 0)),
        ],
        out_specs=[
            pl.BlockSpec((blk, hh), lambda i: (i, 0)),
            pl.BlockSpec((1, hh), lambda i: (0, 0)),
            pl.BlockSpec((1, hh), lambda i: (0, 0)),
        ],
        out_shape=[
            jax.ShapeDtypeStruct((n, hh), jnp.float32),
            jax.ShapeDtypeStruct((1, hh), jnp.float32),
            jax.ShapeDtypeStruct((1, hh), jnp.float32),
        ],
    )(h, parts, wa, ba, wb, bb)


def _bn_body(n, y_ref, sum_ref, sq_ref, g_ref, be_ref, o_ref):
    mean = sum_ref[...] / n
    var = sq_ref[...] / n - mean * mean
    scale = lax.rsqrt(var + 1e-5) * g_ref[...]
    o_ref[...] = jnp.maximum((y_ref[...] - mean) * scale + be_ref[...], 0.0)


def _bn_relu(y, s, q, g, be, blk):
    n, hh = y.shape
    return pl.pallas_call(
        functools.partial(_bn_body, n),
        grid=(n // blk,),
        in_specs=[
            pl.BlockSpec((blk, hh), lambda i: (i, 0)),
            pl.BlockSpec((1, hh), lambda i: (0, 0)),
            pl.BlockSpec((1, hh), lambda i: (0, 0)),
            pl.BlockSpec((1, hh), lambda i: (0, 0)),
            pl.BlockSpec((1, hh), lambda i: (0, 0)),
        ],
        out_specs=pl.BlockSpec((blk, hh), lambda i: (i, 0)),
        out_shape=jax.ShapeDtypeStruct((n, hh), jnp.float32),
    )(y, s, q, g, be)


def _pool_body(n, g, nblk, y_ref, sum_ref, sq_ref, g2_ref, be_ref, b_ref,
               w5_ref, b5_ref, o_ref, acc_ref, cnt_ref):
    i = pl.program_id(0)

    @pl.when(i == 0)
    def _():
        acc_ref[...] = jnp.zeros_like(acc_ref)
        cnt_ref[...] = jnp.zeros_like(cnt_ref)

    mean = sum_ref[...] / n
    var = sq_ref[...] / n - mean * mean
    scale = lax.rsqrt(var + 1e-5) * g2_ref[...]
    h = jnp.maximum((y_ref[...] - mean) * scale + be_ref[...], 0.0)

    mask = (b_ref[...] == lax.broadcasted_iota(jnp.int32, (1, g), 1)
            ).astype(jnp.float32)
    acc_ref[...] += lax.dot_general(mask, h, (((0,), (0,)), ((), ())))
    cnt_ref[...] += jnp.sum(mask, axis=0, keepdims=True)

    @pl.when(i == nblk - 1)
    def _():
        pooled = acc_ref[...] / jnp.maximum(cnt_ref[...], 1.0).reshape(g, 1)
        logits = (jnp.dot(pooled, w5_ref[...],
                          preferred_element_type=jnp.float32) + b5_ref[...])
        m = jnp.max(logits, axis=1, keepdims=True)
        lse = jnp.log(jnp.sum(jnp.exp(logits - m), axis=1, keepdims=True)) + m
        o_ref[...] = logits - lse


def _pool_classify(y, s, q, g2, be2, batch2d, w5p, b5p, g, blk):
    n, hh = y.shape
    cp = w5p.shape[1]
    nblk = n // blk
    return pl.pallas_call(
        functools.partial(_pool_body, n, g, nblk),
        grid=(nblk,),
        in_specs=[
            pl.BlockSpec((blk, hh), lambda i: (i, 0)),
            pl.BlockSpec((1, hh), lambda i: (0, 0)),
            pl.BlockSpec((1, hh), lambda i: (0, 0)),
            pl.BlockSpec((1, hh), lambda i: (0, 0)),
            pl.BlockSpec((1, hh), lambda i: (0, 0)),
            pl.BlockSpec((blk, 1), lambda i: (i, 0)),
            pl.BlockSpec((hh, cp), lambda i: (0, 0)),
            pl.BlockSpec((1, cp), lambda i: (0, 0)),
        ],
        out_specs=pl.BlockSpec((g, cp), lambda i: (0, 0)),
        out_shape=jax.ShapeDtypeStruct((g, cp), jnp.float32),
        scratch_shapes=[
            pltpu.VMEM((g, hh), jnp.float32),
            pltpu.VMEM((1, g), jnp.float32),
        ],
    )(y, s, q, g2, be2, batch2d, w5p, b5p)


def kernel(x, edge_index, batch, W1, b1, W2, b2, g1, be1, W3, b3, W4, b4,
           g2, be2, W5, b5):
    n, d = x.shape
    e = edge_index.shape[1]
    g = 64
    blk = 2000

    nch = -(-e // K)
    nch8 = -(-nch // 8) * 8
    if e != nch8 * K:
        pad = nch8 * K - e
        edge_index = jnp.concatenate(
            [edge_index,
             jnp.stack([jnp.zeros((pad,), jnp.int32),
                        jnp.full((pad,), n, jnp.int32)])], axis=1)
    edge3d = edge_index.reshape(2, nch8, K)
    t0c = min(nch - 1, max(1, int(round(FRAC0 * nch))))
    cb0, cr0 = divmod(t0c, NS)
    cb1, cr1 = divmod(nch - t0c, NS)
    cmax8 = -(-(max(cb0, cb1) + 1 + 7) // 8) * 8
    npad = ((n + NS * 8 - 1) // (NS * 8)) * (NS * 8)
    if npad == n:
        npad += NS * 8
    zblk = jnp.zeros((npad // NS, d), jnp.float32)

    batch2d = batch.reshape(n, 1)
    cpad = 128
    c = W5.shape[1]
    w5p = jnp.pad(W5, ((0, 0), (0, cpad - c)))
    b5p = jnp.pad(b5, (0, cpad - c), constant_values=-1e30).reshape(1, cpad)

    p1 = _sc_agg(x, edge3d, zblk, npad, nch8, cmax8, t0c, cb0, cr0, cb1, cr1)
    y1, s1, q1 = _mlp_stats(x, p1, W1, b1.reshape(1, -1),
                            W2, b2.reshape(1, -1), blk)
    h1 = _bn_relu(y1, s1, q1, g1.reshape(1, -1), be1.reshape(1, -1), blk)

    p2 = _sc_agg(h1, edge3d, zblk, npad, nch8, cmax8, t0c, cb0, cr0, cb1, cr1)
    y2, s2, q2 = _mlp_stats(h1, p2, W3, b3.reshape(1, -1),
                            W4, b4.reshape(1, -1), blk)

    out = _pool_classify(y2, s2, q2, g2.reshape(1, -1), be2.reshape(1, -1),
                         batch2d, w5p, b5p, g, blk)
    return out[:, :c]
